# Initial kernel scaffold; baseline (speedup 1.0000x reference)
import jax
import jax.numpy as jnp
from jax.experimental import pallas as pl


def kernel(x, edge_index, edge_attr, params):
    raise NotImplementedError("write your pallas kernel here")


# speedup vs baseline: 2.1128x; 2.1128x over previous
import functools

import jax
import jax.numpy as jnp
from jax import lax
from jax.experimental import pallas as pl
from jax.experimental.pallas import tpu as pltpu
from jax.experimental.pallas import tpu_sc as plsc

N_NODES_C = 50000
N_EDGES_C = 800000
LATENT_C = 64

_NC = 2
_NS = 16
_NW = _NC * _NS

_CHUNK = 128
_NCHUNK = N_EDGES_C // _CHUNK

@functools.lru_cache(maxsize=None)
def _sc_mesh():
    return plsc.VectorSubcoreMesh(core_axis_name="c", subcore_axis_name="s",
                                  num_cores=_NC, num_subcores=_NS)


@functools.lru_cache(maxsize=None)
def _sc_gather_kernel():
    return functools.partial(
        pl.kernel,
        out_type=(
            jax.ShapeDtypeStruct((N_EDGES_C, LATENT_C), jnp.float32),
            jax.ShapeDtypeStruct((N_EDGES_C, LATENT_C), jnp.float32),
        ),
        mesh=_sc_mesh(),
        scratch_types=[
            pltpu.VMEM((_CHUNK,), jnp.int32),
            pltpu.VMEM((_CHUNK,), jnp.int32),
            pltpu.VMEM((_CHUNK, LATENT_C), jnp.float32),
            pltpu.VMEM((_CHUNK, LATENT_C), jnp.float32),
            pltpu.SemaphoreType.DMA,
            pltpu.SemaphoreType.DMA,
        ],
        compiler_params=pltpu.CompilerParams(use_tc_tiling_on_sc=False),
    )(_sc_gather_body)


def _sc_gather_body(x_hbm, src_hbm, dst_hbm, snd_hbm, rcv_hbm,
                    sidx, didx, srows, drows, sem_a, sem_b):
    wid = lax.axis_index("s") * _NC + lax.axis_index("c")
    n_i = _NCHUNK // _NW + jnp.where(wid < _NCHUNK % _NW, 1, 0)

    def body(i, _):
        k = wid + _NW * i
        base = k * _CHUNK
        pltpu.sync_copy(src_hbm.at[k], sidx)
        pltpu.sync_copy(dst_hbm.at[k], didx)
        cp_a = pltpu.async_copy(x_hbm.at[sidx], srows, sem_a)
        cp_b = pltpu.async_copy(x_hbm.at[didx], drows, sem_b)
        cp_a.wait()
        cp_b.wait()
        pltpu.sync_copy(srows, snd_hbm.at[pl.ds(base, _CHUNK)])
        pltpu.sync_copy(drows, rcv_hbm.at[pl.ds(base, _CHUNK)])
        return ()

    lax.fori_loop(0, n_i, body, ())


_COLS = LATENT_C // _NC
_RPT = N_NODES_C // _NS
_ZC = 125


@functools.lru_cache(maxsize=None)
def _sc_scatter_kernel():
    return functools.partial(
        pl.kernel,
        out_type=jax.ShapeDtypeStruct((N_NODES_C, LATENT_C), jnp.float32),
        mesh=_sc_mesh(),
        scratch_types=[
            pltpu.VMEM((_CHUNK,), jnp.int32),
            pltpu.VMEM((_CHUNK, _COLS), jnp.float32),
            pltpu.VMEM_SHARED((N_NODES_C, _COLS), jnp.float32),
        ],
        compiler_params=pltpu.CompilerParams(use_tc_tiling_on_sc=False),
    )(_sc_scatter_body)


def _sc_scatter_body(upd_hbm, dst_hbm, agg_hbm, idx_v, rows_v, acc_sh):
    c = lax.axis_index("c")
    s = lax.axis_index("s")
    col0 = c * _COLS
    r0 = s * _RPT

    zero16 = jnp.zeros((16,), jnp.float32)

    def zfill(i, _):
        rows_v[i // 2, pl.ds((i % 2) * 16, 16)] = zero16
        return ()

    lax.fori_loop(0, _CHUNK * 2, zfill, ())

    def zcopy(kk, _):
        pltpu.sync_copy(rows_v.at[pl.ds(0, _ZC)],
                        acc_sh.at[pl.ds(r0 + kk * _ZC, _ZC)])
        return ()

    lax.fori_loop(0, _RPT // _ZC, zcopy, ())
    plsc.subcore_barrier()

    n_i = _NCHUNK // _NS + jnp.where(s < _NCHUNK % _NS, 1, 0)

    def abody(i, _):
        k = s + _NS * i
        base = k * _CHUNK
        pltpu.sync_copy(dst_hbm.at[k], idx_v)
        pltpu.sync_copy(upd_hbm.at[pl.ds(base, _CHUNK), pl.ds(col0, _COLS)],
                        rows_v)
        pltpu.sync_copy(rows_v, acc_sh.at[idx_v], add=True)
        return ()

    lax.fori_loop(0, n_i, abody, ())
    plsc.subcore_barrier()

    def wbody(kk, _):
        rr = r0 + kk * _ZC
        pltpu.sync_copy(acc_sh.at[pl.ds(rr, _ZC)], rows_v.at[pl.ds(0, _ZC)])
        pltpu.sync_copy(rows_v.at[pl.ds(0, _ZC)],
                        agg_hbm.at[pl.ds(rr, _ZC), pl.ds(col0, _COLS)])
        return ()

    lax.fori_loop(0, _RPT // _ZC, wbody, ())


_EBLK = 8000


def _edge_mlp_body(s_ref, r_ref, e_ref, w1_ref, b1_ref, w2_ref, b2_ref,
                   g_ref, bb_ref, upd_ref, enew_ref):
    xin = jnp.concatenate([s_ref[...], r_ref[...], e_ref[...]], axis=-1)
    h = jnp.dot(xin, w1_ref[...], preferred_element_type=jnp.float32)
    h = h + b1_ref[...]
    h = h * jax.nn.sigmoid(h)
    o = jnp.dot(h, w2_ref[...], preferred_element_type=jnp.float32)
    o = o + b2_ref[...]
    mu = jnp.mean(o, axis=-1, keepdims=True)
    var = jnp.mean((o - mu) ** 2, axis=-1, keepdims=True)
    on = (o - mu) * lax.rsqrt(var + 1e-5)
    upd = on * g_ref[...] + bb_ref[...]
    upd_ref[...] = upd
    enew_ref[...] = e_ref[...] + upd


def _tc_edge_mlp(snd, rcv, e, w1, b1, w2, b2, g, b):
    grid = (N_EDGES_C // _EBLK,)
    row_spec = pl.BlockSpec((_EBLK, LATENT_C), lambda i: (i, 0))
    full = lambda a: pl.BlockSpec(a.shape, lambda i: (0,) * a.ndim)
    return pl.pallas_call(
        _edge_mlp_body,
        grid=grid,
        in_specs=[row_spec, row_spec, row_spec,
                  full(w1), full(b1), full(w2), full(b2), full(g), full(b)],
        out_specs=[row_spec, row_spec],
        out_shape=[jax.ShapeDtypeStruct((N_EDGES_C, LATENT_C), jnp.float32),
                   jax.ShapeDtypeStruct((N_EDGES_C, LATENT_C), jnp.float32)],
    )(snd, rcv, e, w1, b1, w2, b2, g, b)


_NBLK = 5000


def _node_mlp_body(x_ref, a_ref, w1_ref, b1_ref, w2_ref, b2_ref,
                   g_ref, bb_ref, xnew_ref):
    xin = jnp.concatenate([x_ref[...], a_ref[...]], axis=-1)
    h = jnp.dot(xin, w1_ref[...], preferred_element_type=jnp.float32)
    h = h + b1_ref[...]
    h = h * jax.nn.sigmoid(h)
    o = jnp.dot(h, w2_ref[...], preferred_element_type=jnp.float32)
    o = o + b2_ref[...]
    mu = jnp.mean(o, axis=-1, keepdims=True)
    var = jnp.mean((o - mu) ** 2, axis=-1, keepdims=True)
    on = (o - mu) * lax.rsqrt(var + 1e-5)
    xnew_ref[...] = x_ref[...] + on * g_ref[...] + bb_ref[...]


def _tc_node_mlp(x, agg, w1, b1, w2, b2, g, b):
    grid = (N_NODES_C // _NBLK,)
    row_spec = pl.BlockSpec((_NBLK, LATENT_C), lambda i: (i, 0))
    full = lambda a: pl.BlockSpec(a.shape, lambda i: (0,) * a.ndim)
    return pl.pallas_call(
        _node_mlp_body,
        grid=grid,
        in_specs=[row_spec, row_spec,
                  full(w1), full(b1), full(w2), full(b2), full(g), full(b)],
        out_specs=row_spec,
        out_shape=jax.ShapeDtypeStruct((N_NODES_C, LATENT_C), jnp.float32),
    )(x, agg, w1, b1, w2, b2, g, b)


def kernel(x, edge_index, edge_attr, params):
    src2 = edge_index[0].astype(jnp.int32).reshape(_NCHUNK, _CHUNK)
    dst2 = edge_index[1].astype(jnp.int32).reshape(_NCHUNK, _CHUNK)
    row2 = lambda a: a.reshape(1, -1)
    for lp in params:
        ep, np_ = lp['edge'], lp['node']
        snd, rcv = _sc_gather_kernel()(x, src2, dst2)
        upd, e_new = _tc_edge_mlp(snd, rcv, edge_attr,
                                  ep['W1'], row2(ep['b1']),
                                  ep['W2'], row2(ep['b2']),
                                  row2(ep['g']), row2(ep['b']))
        agg = _sc_scatter_kernel()(upd, dst2)
        x = _tc_node_mlp(x, agg,
                         np_['W1'], row2(np_['b1']),
                         np_['W2'], row2(np_['b2']),
                         row2(np_['g']), row2(np_['b']))
        edge_attr = e_new
    return (x, edge_attr)


# speedup vs baseline: 2.7632x; 1.3079x over previous
import functools

import jax
import jax.numpy as jnp
from jax import lax
from jax.experimental import pallas as pl
from jax.experimental.pallas import tpu as pltpu
from jax.experimental.pallas import tpu_sc as plsc

N_NODES_C = 50000
N_EDGES_C = 800000
LATENT_C = 64

_NC = 2
_NS = 16
_NW = _NC * _NS

_IDXW = 125


@functools.lru_cache(maxsize=None)
def _sc_mesh():
    return plsc.VectorSubcoreMesh(core_axis_name="c", subcore_axis_name="s",
                                  num_cores=_NC, num_subcores=_NS)


_E_W = N_EDGES_C // _NW
_G = 250
_NG = _E_W // _G


@functools.lru_cache(maxsize=None)
def _sc_gather_kernel():
    return functools.partial(
        pl.kernel,
        out_type=(
            jax.ShapeDtypeStruct((N_EDGES_C, LATENT_C), jnp.float32),
            jax.ShapeDtypeStruct((N_EDGES_C, LATENT_C), jnp.float32),
        ),
        mesh=_sc_mesh(),
        scratch_types=[
            pltpu.VMEM((_E_W // _IDXW, _IDXW), jnp.int32),
            pltpu.VMEM((_E_W // _IDXW, _IDXW), jnp.int32),
            [pltpu.VMEM((_G, LATENT_C), jnp.float32) for _ in range(2)],
            [pltpu.VMEM((_G, LATENT_C), jnp.float32) for _ in range(2)],
            [pltpu.SemaphoreType.DMA for _ in range(2)],
            [pltpu.SemaphoreType.DMA for _ in range(2)],
            [pltpu.SemaphoreType.DMA for _ in range(2)],
            [pltpu.SemaphoreType.DMA for _ in range(2)],
        ],
        compiler_params=pltpu.CompilerParams(use_tc_tiling_on_sc=False),
    )(_sc_gather_body)


def _sc_gather_body(x_hbm, src_hbm, dst_hbm, snd_hbm, rcv_hbm,
                    sidx, didx, srows, drows, gsem, dsem, osem_s, osem_r):
    wid = lax.axis_index("s") * _NC + lax.axis_index("c")
    e0 = wid * _E_W
    nrow = _E_W // _IDXW
    pltpu.sync_copy(src_hbm.at[pl.ds(wid * nrow, nrow)], sidx)
    pltpu.sync_copy(dst_hbm.at[pl.ds(wid * nrow, nrow)], didx)

    def fire(g, b):
        for j in range(_G // _IDXW):
            kk = g * (_G // _IDXW) + j
            pltpu.async_copy(x_hbm.at[sidx.at[kk]],
                             srows[b].at[pl.ds(j * _IDXW, _IDXW)], gsem[b])
            pltpu.async_copy(x_hbm.at[didx.at[kk]],
                             drows[b].at[pl.ds(j * _IDXW, _IDXW)], dsem[b])

    def drain_gathers(b):
        for j in range(_G // _IDXW):
            pltpu.make_async_copy(
                x_hbm.at[sidx.at[0]],
                srows[b].at[pl.ds(0, _IDXW)], gsem[b]).wait()
            pltpu.make_async_copy(
                x_hbm.at[didx.at[0]],
                drows[b].at[pl.ds(0, _IDXW)], dsem[b]).wait()

    def writeback(g, b):
        base = e0 + g * _G
        pltpu.async_copy(srows[b], snd_hbm.at[pl.ds(base, _G)], osem_s[b])
        pltpu.async_copy(drows[b], rcv_hbm.at[pl.ds(base, _G)], osem_r[b])

    def wait_writeback(b):
        pltpu.make_async_copy(srows[b], snd_hbm.at[pl.ds(0, _G)],
                              osem_s[b]).wait()
        pltpu.make_async_copy(drows[b], rcv_hbm.at[pl.ds(0, _G)],
                              osem_r[b]).wait()

    def body(i, _):
        for b in range(2):
            g = 2 * i + b

            @pl.when(g >= 2)
            def _():
                wait_writeback(b)

            fire(g, b)

            @pl.when(g >= 1)
            def _():
                drain_gathers(1 - b)
                writeback(g - 1, 1 - b)

        return ()

    lax.fori_loop(0, _NG // 2, body, ())
    drain_gathers(1)
    writeback(_NG - 1, 1)
    wait_writeback(0)
    wait_writeback(1)


_COLS = LATENT_C // _NC
_E_T = N_EDGES_C // _NS
_G2 = 250
_NG2 = _E_T // _G2
_RPT = N_NODES_C // _NS
_GIR = _G2 // _IDXW


@functools.lru_cache(maxsize=None)
def _sc_scatter_kernel():
    return functools.partial(
        pl.kernel,
        out_type=jax.ShapeDtypeStruct((N_NODES_C, LATENT_C), jnp.float32),
        mesh=_sc_mesh(),
        scratch_types=[
            [pltpu.VMEM((_GIR, _IDXW), jnp.int32) for _ in range(2)],
            [pltpu.VMEM((_G2, _COLS), jnp.float32) for _ in range(2)],
            [pltpu.SemaphoreType.DMA for _ in range(2)],
            [pltpu.SemaphoreType.DMA for _ in range(2)],
            pltpu.VMEM_SHARED((N_NODES_C, _COLS), jnp.float32),
        ],
        compiler_params=pltpu.CompilerParams(use_tc_tiling_on_sc=False),
    )(_sc_scatter_body)


def _sc_scatter_body(upd_hbm, dst_hbm, agg_hbm, idx2, rows, rsem, isem,
                     acc_sh):
    c = lax.axis_index("c")
    s = lax.axis_index("s")
    col0 = c * _COLS
    r0 = s * _RPT
    e0 = s * _E_T
    i0 = s * (_E_T // _IDXW)

    zero16 = jnp.zeros((16,), jnp.float32)

    def zfill(i, _):
        rows[0][i // 2, pl.ds((i % 2) * 16, 16)] = zero16
        return ()

    lax.fori_loop(0, _G2 * 2, zfill, ())

    nfull = _RPT // _G2
    rem = _RPT - nfull * _G2

    def zcopy(kk, _):
        pltpu.sync_copy(rows[0], acc_sh.at[pl.ds(r0 + kk * _G2, _G2)])
        return ()

    lax.fori_loop(0, nfull, zcopy, ())
    pltpu.sync_copy(rows[0].at[pl.ds(0, rem)],
                    acc_sh.at[pl.ds(r0 + nfull * _G2, rem)])
    plsc.subcore_barrier()

    def fire(g, b):
        base = e0 + g * _G2
        pltpu.async_copy(dst_hbm.at[pl.ds(i0 + g * _GIR, _GIR)],
                         idx2[b], isem[b])
        pltpu.async_copy(upd_hbm.at[pl.ds(base, _G2), pl.ds(col0, _COLS)],
                         rows[b], rsem[b])

    def complete(b):
        pltpu.make_async_copy(dst_hbm.at[pl.ds(0, _GIR)],
                              idx2[b], isem[b]).wait()
        pltpu.make_async_copy(
            upd_hbm.at[pl.ds(0, _G2), pl.ds(col0, _COLS)],
            rows[b], rsem[b]).wait()
        for j in range(_GIR):
            pltpu.sync_copy(rows[b].at[pl.ds(j * _IDXW, _IDXW)],
                            acc_sh.at[idx2[b].at[j]], add=True)

    def body(i, _):
        for b in range(2):
            g = 2 * i + b
            fire(g, b)

            @pl.when(g >= 1)
            def _():
                complete(1 - b)

        return ()

    lax.fori_loop(0, _NG2 // 2, body, ())
    complete(1)
    plsc.subcore_barrier()

    pltpu.sync_copy(acc_sh.at[pl.ds(r0, _RPT)],
                    agg_hbm.at[pl.ds(r0, _RPT), pl.ds(col0, _COLS)])


_EBLK = 8000


def _edge_mlp_body(s_ref, r_ref, e_ref, w1_ref, b1_ref, w2_ref, b2_ref,
                   g_ref, bb_ref, upd_ref, enew_ref):
    xin = jnp.concatenate([s_ref[...], r_ref[...], e_ref[...]], axis=-1)
    h = jnp.dot(xin, w1_ref[...], preferred_element_type=jnp.float32)
    h = h + b1_ref[...]
    h = h * jax.nn.sigmoid(h)
    o = jnp.dot(h, w2_ref[...], preferred_element_type=jnp.float32)
    o = o + b2_ref[...]
    mu = jnp.mean(o, axis=-1, keepdims=True)
    var = jnp.mean((o - mu) ** 2, axis=-1, keepdims=True)
    on = (o - mu) * lax.rsqrt(var + 1e-5)
    upd = on * g_ref[...] + bb_ref[...]
    upd_ref[...] = upd
    enew_ref[...] = e_ref[...] + upd


def _tc_edge_mlp(snd, rcv, e, w1, b1, w2, b2, g, b):
    grid = (N_EDGES_C // _EBLK,)
    row_spec = pl.BlockSpec((_EBLK, LATENT_C), lambda i: (i, 0))
    full = lambda a: pl.BlockSpec(a.shape, lambda i: (0,) * a.ndim)
    return pl.pallas_call(
        _edge_mlp_body,
        grid=grid,
        in_specs=[row_spec, row_spec, row_spec,
                  full(w1), full(b1), full(w2), full(b2), full(g), full(b)],
        out_specs=[row_spec, row_spec],
        out_shape=[jax.ShapeDtypeStruct((N_EDGES_C, LATENT_C), jnp.float32),
                   jax.ShapeDtypeStruct((N_EDGES_C, LATENT_C), jnp.float32)],
    )(snd, rcv, e, w1, b1, w2, b2, g, b)


_NBLK = 5000


def _node_mlp_body(x_ref, a_ref, w1_ref, b1_ref, w2_ref, b2_ref,
                   g_ref, bb_ref, xnew_ref):
    xin = jnp.concatenate([x_ref[...], a_ref[...]], axis=-1)
    h = jnp.dot(xin, w1_ref[...], preferred_element_type=jnp.float32)
    h = h + b1_ref[...]
    h = h * jax.nn.sigmoid(h)
    o = jnp.dot(h, w2_ref[...], preferred_element_type=jnp.float32)
    o = o + b2_ref[...]
    mu = jnp.mean(o, axis=-1, keepdims=True)
    var = jnp.mean((o - mu) ** 2, axis=-1, keepdims=True)
    on = (o - mu) * lax.rsqrt(var + 1e-5)
    xnew_ref[...] = x_ref[...] + on * g_ref[...] + bb_ref[...]


def _tc_node_mlp(x, agg, w1, b1, w2, b2, g, b):
    grid = (N_NODES_C // _NBLK,)
    row_spec = pl.BlockSpec((_NBLK, LATENT_C), lambda i: (i, 0))
    full = lambda a: pl.BlockSpec(a.shape, lambda i: (0,) * a.ndim)
    return pl.pallas_call(
        _node_mlp_body,
        grid=grid,
        in_specs=[row_spec, row_spec,
                  full(w1), full(b1), full(w2), full(b2), full(g), full(b)],
        out_specs=row_spec,
        out_shape=jax.ShapeDtypeStruct((N_NODES_C, LATENT_C), jnp.float32),
    )(x, agg, w1, b1, w2, b2, g, b)


def kernel(x, edge_index, edge_attr, params):
    src2 = edge_index[0].astype(jnp.int32).reshape(N_EDGES_C // _IDXW, _IDXW)
    dst2 = edge_index[1].astype(jnp.int32).reshape(N_EDGES_C // _IDXW, _IDXW)
    row2 = lambda a: a.reshape(1, -1)
    for lp in params:
        ep, np_ = lp['edge'], lp['node']
        snd, rcv = _sc_gather_kernel()(x, src2, dst2)
        upd, e_new = _tc_edge_mlp(snd, rcv, edge_attr,
                                  ep['W1'], row2(ep['b1']),
                                  ep['W2'], row2(ep['b2']),
                                  row2(ep['g']), row2(ep['b']))
        agg = _sc_scatter_kernel()(upd, dst2)
        x = _tc_node_mlp(x, agg,
                         np_['W1'], row2(np_['b1']),
                         np_['W2'], row2(np_['b2']),
                         row2(np_['g']), row2(np_['b']))
        edge_attr = e_new
    return (x, edge_attr)


# speedup vs baseline: 5.2831x; 1.9120x over previous
import functools

import jax
import jax.numpy as jnp
from jax import lax
from jax.experimental import pallas as pl
from jax.experimental.pallas import tpu as pltpu
from jax.experimental.pallas import tpu_sc as plsc

N_NODES_C = 50000
N_EDGES_C = 800000
LATENT_C = 64

_NC = 2
_NS = 16
_NW = _NC * _NS

_IDXW = 125


@functools.lru_cache(maxsize=None)
def _sc_mesh():
    return plsc.VectorSubcoreMesh(core_axis_name="c", subcore_axis_name="s",
                                  num_cores=_NC, num_subcores=_NS)


_E_W = N_EDGES_C // _NW
_G = 250
_NG = _E_W // _G


@functools.lru_cache(maxsize=None)
def _sc_gather_kernel():
    return functools.partial(
        pl.kernel,
        out_type=jax.ShapeDtypeStruct((N_EDGES_C, 2 * LATENT_C), jnp.float32),
        mesh=_sc_mesh(),
        scratch_types=[
            pltpu.VMEM((_E_W // _IDXW, _IDXW), jnp.int32),
            pltpu.VMEM((_E_W // _IDXW, _IDXW), jnp.int32),
            [pltpu.VMEM((_G, LATENT_C), jnp.float32) for _ in range(2)],
            [pltpu.VMEM((_G, LATENT_C), jnp.float32) for _ in range(2)],
            [pltpu.SemaphoreType.DMA for _ in range(2)],
            [pltpu.SemaphoreType.DMA for _ in range(2)],
            [pltpu.SemaphoreType.DMA for _ in range(2)],
            [pltpu.SemaphoreType.DMA for _ in range(2)],
        ],
        compiler_params=pltpu.CompilerParams(use_tc_tiling_on_sc=False),
    )(_sc_gather_body)


def _sc_gather_body(x_hbm, src_hbm, dst_hbm, g2_hbm,
                    sidx, didx, srows, drows, gsem, dsem, osem_s, osem_r):
    wid = lax.axis_index("s") * _NC + lax.axis_index("c")
    e0 = wid * _E_W
    nrow = _E_W // _IDXW
    pltpu.sync_copy(src_hbm.at[pl.ds(wid * nrow, nrow)], sidx)
    pltpu.sync_copy(dst_hbm.at[pl.ds(wid * nrow, nrow)], didx)

    def fire(g, b):
        for j in range(_G // _IDXW):
            kk = g * (_G // _IDXW) + j
            pltpu.async_copy(x_hbm.at[sidx.at[kk]],
                             srows[b].at[pl.ds(j * _IDXW, _IDXW)], gsem[b])
            pltpu.async_copy(x_hbm.at[didx.at[kk]],
                             drows[b].at[pl.ds(j * _IDXW, _IDXW)], dsem[b])

    def drain_gathers(b):
        for j in range(_G // _IDXW):
            pltpu.make_async_copy(
                x_hbm.at[sidx.at[0]],
                srows[b].at[pl.ds(0, _IDXW)], gsem[b]).wait()
            pltpu.make_async_copy(
                x_hbm.at[didx.at[0]],
                drows[b].at[pl.ds(0, _IDXW)], dsem[b]).wait()

    def writeback(g, b):
        base = e0 + g * _G
        pltpu.async_copy(
            srows[b], g2_hbm.at[pl.ds(base, _G), pl.ds(0, LATENT_C)],
            osem_s[b])
        pltpu.async_copy(
            drows[b], g2_hbm.at[pl.ds(base, _G), pl.ds(LATENT_C, LATENT_C)],
            osem_r[b])

    def wait_writeback(b):
        pltpu.make_async_copy(
            srows[b], g2_hbm.at[pl.ds(0, _G), pl.ds(0, LATENT_C)],
            osem_s[b]).wait()
        pltpu.make_async_copy(
            drows[b], g2_hbm.at[pl.ds(0, _G), pl.ds(LATENT_C, LATENT_C)],
            osem_r[b]).wait()

    def body(i, _):
        for b in range(2):
            g = 2 * i + b

            @pl.when(g >= 2)
            def _():
                wait_writeback(b)

            fire(g, b)

            @pl.when(g >= 1)
            def _():
                drain_gathers(1 - b)
                writeback(g - 1, 1 - b)

        return ()

    lax.fori_loop(0, _NG // 2, body, ())
    drain_gathers(1)
    writeback(_NG - 1, 1)
    wait_writeback(0)
    wait_writeback(1)


_COLS = LATENT_C // _NC
_E_T = N_EDGES_C // _NS
_G2 = 250
_NG2 = _E_T // _G2
_RPT = N_NODES_C // _NS
_GIR = _G2 // _IDXW


@functools.lru_cache(maxsize=None)
def _sc_scatter_kernel():
    return functools.partial(
        pl.kernel,
        out_type=jax.ShapeDtypeStruct((N_NODES_C, LATENT_C), jnp.float32),
        mesh=_sc_mesh(),
        scratch_types=[
            [pltpu.VMEM((_GIR, _IDXW), jnp.int32) for _ in range(2)],
            [pltpu.VMEM((_G2, _COLS), jnp.float32) for _ in range(2)],
            [pltpu.SemaphoreType.DMA for _ in range(2)],
            [pltpu.SemaphoreType.DMA for _ in range(2)],
            pltpu.VMEM_SHARED((N_NODES_C, _COLS), jnp.float32),
        ],
        compiler_params=pltpu.CompilerParams(use_tc_tiling_on_sc=False),
    )(_sc_scatter_body)


def _sc_scatter_body(upd_hbm, dst_hbm, agg_hbm, idx2, rows, rsem, isem,
                     acc_sh):
    c = lax.axis_index("c")
    s = lax.axis_index("s")
    col0 = c * _COLS
    r0 = s * _RPT
    e0 = s * _E_T
    i0 = s * (_E_T // _IDXW)

    zero16 = jnp.zeros((16,), jnp.float32)

    def zfill(i, _):
        rows[0][i // 2, pl.ds((i % 2) * 16, 16)] = zero16
        return ()

    lax.fori_loop(0, _G2 * 2, zfill, ())

    nfull = _RPT // _G2
    rem = _RPT - nfull * _G2

    def zcopy(kk, _):
        pltpu.sync_copy(rows[0], acc_sh.at[pl.ds(r0 + kk * _G2, _G2)])
        return ()

    lax.fori_loop(0, nfull, zcopy, ())
    pltpu.sync_copy(rows[0].at[pl.ds(0, rem)],
                    acc_sh.at[pl.ds(r0 + nfull * _G2, rem)])
    plsc.subcore_barrier()

    def fire(g, b):
        base = e0 + g * _G2
        pltpu.async_copy(dst_hbm.at[pl.ds(i0 + g * _GIR, _GIR)],
                         idx2[b], isem[b])
        pltpu.async_copy(upd_hbm.at[pl.ds(base, _G2), pl.ds(col0, _COLS)],
                         rows[b], rsem[b])

    def complete(b):
        pltpu.make_async_copy(dst_hbm.at[pl.ds(0, _GIR)],
                              idx2[b], isem[b]).wait()
        pltpu.make_async_copy(
            upd_hbm.at[pl.ds(0, _G2), pl.ds(col0, _COLS)],
            rows[b], rsem[b]).wait()
        for j in range(_GIR):
            pltpu.sync_copy(rows[b].at[pl.ds(j * _IDXW, _IDXW)],
                            acc_sh.at[idx2[b].at[j]], add=True)

    def body(i, _):
        for b in range(2):
            g = 2 * i + b
            fire(g, b)

            @pl.when(g >= 1)
            def _():
                complete(1 - b)

        return ()

    lax.fori_loop(0, _NG2 // 2, body, ())
    complete(1)
    plsc.subcore_barrier()

    pltpu.sync_copy(acc_sh.at[pl.ds(r0, _RPT)],
                    agg_hbm.at[pl.ds(r0, _RPT), pl.ds(col0, _COLS)])


_EBLK = 8000


def _edge_core(xin, e, w1_ref, b1_ref, w2_ref, b2_ref, g_ref, bb_ref,
               last):
    h = jnp.dot(xin, w1_ref[...], preferred_element_type=jnp.float32)
    h = h + b1_ref[...]
    h = h * jax.nn.sigmoid(h)
    o = jnp.dot(h, w2_ref[...], preferred_element_type=jnp.float32)
    o = o + b2_ref[...]
    mu = jnp.mean(o, axis=-1, keepdims=True)
    var = jnp.mean((o - mu) ** 2, axis=-1, keepdims=True)
    on = (o - mu) * lax.rsqrt(var + 1e-5)
    upd = on * g_ref[...] + bb_ref[...]
    if last:
        return upd, e + upd
    return jnp.concatenate([upd, e + upd], axis=-1), None


def _edge_mlp_first_body(g2_ref, e_ref, w1_ref, b1_ref, w2_ref, b2_ref,
                         g_ref, bb_ref, upd2_ref):
    e = e_ref[...]
    xin = jnp.concatenate([g2_ref[...], e], axis=-1)
    upd2_ref[...], _ = _edge_core(xin, e, w1_ref, b1_ref, w2_ref, b2_ref,
                                  g_ref, bb_ref, last=False)


def _edge_mlp_mid_body(g2_ref, p2_ref, w1_ref, b1_ref, w2_ref, b2_ref,
                       g_ref, bb_ref, upd2_ref):
    e = p2_ref[:, LATENT_C:]
    xin = jnp.concatenate([g2_ref[...], e], axis=-1)
    upd2_ref[...], _ = _edge_core(xin, e, w1_ref, b1_ref, w2_ref, b2_ref,
                                  g_ref, bb_ref, last=False)


def _edge_mlp_last_body(g2_ref, p2_ref, w1_ref, b1_ref, w2_ref, b2_ref,
                        g_ref, bb_ref, upd2_ref, enew_ref):
    e = p2_ref[:, LATENT_C:]
    xin = jnp.concatenate([g2_ref[...], e], axis=-1)
    upd, enew = _edge_core(xin, e, w1_ref, b1_ref, w2_ref, b2_ref,
                           g_ref, bb_ref, last=True)
    upd2_ref[...] = jnp.concatenate([upd, upd], axis=-1)
    enew_ref[...] = enew


def _tc_edge_mlp(kind, g2, e, w1, b1, w2, b2, g, b):
    grid = (N_EDGES_C // _EBLK,)
    wide = pl.BlockSpec((_EBLK, 2 * LATENT_C), lambda i: (i, 0))
    e_spec = pl.BlockSpec((_EBLK, e.shape[1]), lambda i: (i, 0))
    full = lambda a: pl.BlockSpec(a.shape, lambda i: (0,) * a.ndim)
    body = {"first": _edge_mlp_first_body, "mid": _edge_mlp_mid_body,
            "last": _edge_mlp_last_body}[kind]
    wide_out = jax.ShapeDtypeStruct((N_EDGES_C, 2 * LATENT_C), jnp.float32)
    if kind == "last":
        out_specs = [wide, pl.BlockSpec((_EBLK, LATENT_C), lambda i: (i, 0))]
        out_shape = [wide_out,
                     jax.ShapeDtypeStruct((N_EDGES_C, LATENT_C), jnp.float32)]
    else:
        out_specs = wide
        out_shape = wide_out
    return pl.pallas_call(
        body,
        grid=grid,
        in_specs=[wide, e_spec,
                  full(w1), full(b1), full(w2), full(b2), full(g), full(b)],
        out_specs=out_specs,
        out_shape=out_shape,
    )(g2, e, w1, b1, w2, b2, g, b)


_NBLK = 5000


def _node_mlp_body(x_ref, a_ref, w1_ref, b1_ref, w2_ref, b2_ref,
                   g_ref, bb_ref, xnew_ref):
    xin = jnp.concatenate([x_ref[...], a_ref[...]], axis=-1)
    h = jnp.dot(xin, w1_ref[...], preferred_element_type=jnp.float32)
    h = h + b1_ref[...]
    h = h * jax.nn.sigmoid(h)
    o = jnp.dot(h, w2_ref[...], preferred_element_type=jnp.float32)
    o = o + b2_ref[...]
    mu = jnp.mean(o, axis=-1, keepdims=True)
    var = jnp.mean((o - mu) ** 2, axis=-1, keepdims=True)
    on = (o - mu) * lax.rsqrt(var + 1e-5)
    xnew_ref[...] = x_ref[...] + on * g_ref[...] + bb_ref[...]


def _tc_node_mlp(x, agg, w1, b1, w2, b2, g, b):
    grid = (N_NODES_C // _NBLK,)
    row_spec = pl.BlockSpec((_NBLK, LATENT_C), lambda i: (i, 0))
    full = lambda a: pl.BlockSpec(a.shape, lambda i: (0,) * a.ndim)
    return pl.pallas_call(
        _node_mlp_body,
        grid=grid,
        in_specs=[row_spec, row_spec,
                  full(w1), full(b1), full(w2), full(b2), full(g), full(b)],
        out_specs=row_spec,
        out_shape=jax.ShapeDtypeStruct((N_NODES_C, LATENT_C), jnp.float32),
    )(x, agg, w1, b1, w2, b2, g, b)


def kernel(x, edge_index, edge_attr, params):
    src2 = edge_index[0].astype(jnp.int32).reshape(N_EDGES_C // _IDXW, _IDXW)
    dst2 = edge_index[1].astype(jnp.int32).reshape(N_EDGES_C // _IDXW, _IDXW)
    row2 = lambda a: a.reshape(1, -1)
    n_layers = len(params)
    prev2 = None
    e_new = None
    for li, lp in enumerate(params):
        ep, np_ = lp['edge'], lp['node']
        g2 = _sc_gather_kernel()(x, src2, dst2)
        kind = ("first" if li == 0 else
                "last" if li == n_layers - 1 else "mid")
        e_arg = edge_attr if li == 0 else prev2
        res = _tc_edge_mlp(kind, g2, e_arg,
                           ep['W1'], row2(ep['b1']),
                           ep['W2'], row2(ep['b2']),
                           row2(ep['g']), row2(ep['b']))
        if kind == "last":
            upd2, e_new = res
        else:
            upd2 = res
        agg = _sc_scatter_kernel()(upd2, dst2)
        x = _tc_node_mlp(x, agg,
                         np_['W1'], row2(np_['b1']),
                         np_['W2'], row2(np_['b2']),
                         row2(np_['g']), row2(np_['b']))
        prev2 = upd2
    return (x, e_new)


# speedup vs baseline: 5.3755x; 1.0175x over previous
import functools

import jax
import jax.numpy as jnp
from jax import lax
from jax.experimental import pallas as pl
from jax.experimental.pallas import tpu as pltpu
from jax.experimental.pallas import tpu_sc as plsc

N_NODES_C = 50000
N_EDGES_C = 800000
LATENT_C = 64

_NC = 2
_NS = 16
_NW = _NC * _NS

_IDXW = 125


@functools.lru_cache(maxsize=None)
def _sc_mesh():
    return plsc.VectorSubcoreMesh(core_axis_name="c", subcore_axis_name="s",
                                  num_cores=_NC, num_subcores=_NS)


_E_W = N_EDGES_C // _NW
_G = 250
_NG = _E_W // _G


@functools.lru_cache(maxsize=None)
def _sc_gather_kernel():
    return functools.partial(
        pl.kernel,
        out_type=jax.ShapeDtypeStruct((N_EDGES_C, 2 * LATENT_C), jnp.float32),
        mesh=_sc_mesh(),
        scratch_types=[
            pltpu.VMEM((_E_W // _IDXW, _IDXW), jnp.int32),
            pltpu.VMEM((_E_W // _IDXW, _IDXW), jnp.int32),
            [pltpu.VMEM((_G, LATENT_C), jnp.float32) for _ in range(2)],
            [pltpu.VMEM((_G, LATENT_C), jnp.float32) for _ in range(2)],
            [pltpu.SemaphoreType.DMA for _ in range(2)],
            [pltpu.SemaphoreType.DMA for _ in range(2)],
            [pltpu.SemaphoreType.DMA for _ in range(2)],
            [pltpu.SemaphoreType.DMA for _ in range(2)],
        ],
        compiler_params=pltpu.CompilerParams(use_tc_tiling_on_sc=False),
    )(_sc_gather_body)


def _sc_gather_body(x_hbm, src_hbm, dst_hbm, g2_hbm,
                    sidx, didx, srows, drows, gsem, dsem, osem_s, osem_r):
    wid = lax.axis_index("s") * _NC + lax.axis_index("c")
    e0 = wid * _E_W
    nrow = _E_W // _IDXW
    pltpu.sync_copy(src_hbm.at[pl.ds(wid * nrow, nrow)], sidx)
    pltpu.sync_copy(dst_hbm.at[pl.ds(wid * nrow, nrow)], didx)

    def fire(g, b):
        for j in range(_G // _IDXW):
            kk = g * (_G // _IDXW) + j
            pltpu.async_copy(x_hbm.at[sidx.at[kk]],
                             srows[b].at[pl.ds(j * _IDXW, _IDXW)], gsem[b])
            pltpu.async_copy(x_hbm.at[didx.at[kk]],
                             drows[b].at[pl.ds(j * _IDXW, _IDXW)], dsem[b])

    def drain_gathers(b):
        for j in range(_G // _IDXW):
            pltpu.make_async_copy(
                x_hbm.at[sidx.at[0]],
                srows[b].at[pl.ds(0, _IDXW)], gsem[b]).wait()
            pltpu.make_async_copy(
                x_hbm.at[didx.at[0]],
                drows[b].at[pl.ds(0, _IDXW)], dsem[b]).wait()

    def writeback(g, b):
        base = e0 + g * _G
        pltpu.async_copy(
            srows[b], g2_hbm.at[pl.ds(base, _G), pl.ds(0, LATENT_C)],
            osem_s[b])
        pltpu.async_copy(
            drows[b], g2_hbm.at[pl.ds(base, _G), pl.ds(LATENT_C, LATENT_C)],
            osem_r[b])

    def wait_writeback(b):
        pltpu.make_async_copy(
            srows[b], g2_hbm.at[pl.ds(0, _G), pl.ds(0, LATENT_C)],
            osem_s[b]).wait()
        pltpu.make_async_copy(
            drows[b], g2_hbm.at[pl.ds(0, _G), pl.ds(LATENT_C, LATENT_C)],
            osem_r[b]).wait()

    def body(i, _):
        for b in range(2):
            g = 2 * i + b

            @pl.when(g >= 2)
            def _():
                wait_writeback(b)

            fire(g, b)

            @pl.when(g >= 1)
            def _():
                drain_gathers(1 - b)
                writeback(g - 1, 1 - b)

        return ()

    lax.fori_loop(0, _NG // 2, body, ())
    drain_gathers(1)
    writeback(_NG - 1, 1)
    wait_writeback(0)
    wait_writeback(1)


_COLS = LATENT_C // _NC
_E_T = N_EDGES_C // _NS
_G2 = 250
_NG2 = _E_T // _G2
_RPT = N_NODES_C // _NS
_GIR = _G2 // _IDXW


@functools.lru_cache(maxsize=None)
def _sc_scatter_kernel():
    return functools.partial(
        pl.kernel,
        out_type=jax.ShapeDtypeStruct((N_NODES_C, LATENT_C), jnp.float32),
        mesh=_sc_mesh(),
        scratch_types=[
            [pltpu.VMEM((_GIR, _IDXW), jnp.int32) for _ in range(2)],
            [pltpu.VMEM((_G2, _COLS), jnp.float32) for _ in range(2)],
            [pltpu.SemaphoreType.DMA for _ in range(2)],
            [pltpu.SemaphoreType.DMA for _ in range(2)],
            pltpu.VMEM_SHARED((N_NODES_C, _COLS), jnp.float32),
        ],
        compiler_params=pltpu.CompilerParams(use_tc_tiling_on_sc=False),
    )(_sc_scatter_body)


def _sc_scatter_body(upd_hbm, dst_hbm, agg_hbm, idx2, rows, rsem, isem,
                     acc_sh):
    c = lax.axis_index("c")
    s = lax.axis_index("s")
    col0 = c * _COLS
    r0 = s * _RPT
    e0 = s * _E_T
    i0 = s * (_E_T // _IDXW)

    zero16 = jnp.zeros((16,), jnp.float32)

    def zfill(i, _):
        rows[0][i // 2, pl.ds((i % 2) * 16, 16)] = zero16
        return ()

    lax.fori_loop(0, _G2 * 2, zfill, ())

    nfull = _RPT // _G2
    rem = _RPT - nfull * _G2

    def zcopy(kk, _):
        pltpu.sync_copy(rows[0], acc_sh.at[pl.ds(r0 + kk * _G2, _G2)])
        return ()

    lax.fori_loop(0, nfull, zcopy, ())
    pltpu.sync_copy(rows[0].at[pl.ds(0, rem)],
                    acc_sh.at[pl.ds(r0 + nfull * _G2, rem)])
    plsc.subcore_barrier()

    def fire(g, b):
        base = e0 + g * _G2
        pltpu.async_copy(dst_hbm.at[pl.ds(i0 + g * _GIR, _GIR)],
                         idx2[b], isem[b])
        pltpu.async_copy(upd_hbm.at[pl.ds(base, _G2), pl.ds(col0, _COLS)],
                         rows[b], rsem[b])

    def complete(b):
        pltpu.make_async_copy(dst_hbm.at[pl.ds(0, _GIR)],
                              idx2[b], isem[b]).wait()
        pltpu.make_async_copy(
            upd_hbm.at[pl.ds(0, _G2), pl.ds(col0, _COLS)],
            rows[b], rsem[b]).wait()
        for j in range(_GIR):
            pltpu.sync_copy(rows[b].at[pl.ds(j * _IDXW, _IDXW)],
                            acc_sh.at[idx2[b].at[j]], add=True)

    def body(i, _):
        for b in range(2):
            g = 2 * i + b
            fire(g, b)

            @pl.when(g >= 1)
            def _():
                complete(1 - b)

        return ()

    lax.fori_loop(0, _NG2 // 2, body, ())
    complete(1)
    plsc.subcore_barrier()

    pltpu.sync_copy(acc_sh.at[pl.ds(r0, _RPT)],
                    agg_hbm.at[pl.ds(r0, _RPT), pl.ds(col0, _COLS)])


_EBLK = 10000


def _edge_core(xin, e, w1_ref, b1_ref, w2_ref, b2_ref, g_ref, bb_ref,
               last):
    h = jnp.dot(xin, w1_ref[...], preferred_element_type=jnp.float32)
    h = h + b1_ref[...]
    h = h * jax.nn.sigmoid(h)
    o = jnp.dot(h, w2_ref[...], preferred_element_type=jnp.float32)
    o = o + b2_ref[...]
    mu = jnp.mean(o, axis=-1, keepdims=True)
    var = jnp.mean((o - mu) ** 2, axis=-1, keepdims=True)
    on = (o - mu) * lax.rsqrt(var + 1e-5)
    upd = on * g_ref[...] + bb_ref[...]
    if last:
        return upd, e + upd
    return jnp.concatenate([upd, e + upd], axis=-1), None


def _edge_mlp_first_body(g2_ref, e_ref, w1_ref, b1_ref, w2_ref, b2_ref,
                         g_ref, bb_ref, upd2_ref):
    e = e_ref[...]
    xin = jnp.concatenate([g2_ref[...], e], axis=-1)
    upd2_ref[...], _ = _edge_core(xin, e, w1_ref, b1_ref, w2_ref, b2_ref,
                                  g_ref, bb_ref, last=False)


def _edge_mlp_mid_body(g2_ref, p2_ref, w1_ref, b1_ref, w2_ref, b2_ref,
                       g_ref, bb_ref, upd2_ref):
    e = p2_ref[:, LATENT_C:]
    xin = jnp.concatenate([g2_ref[...], e], axis=-1)
    upd2_ref[...], _ = _edge_core(xin, e, w1_ref, b1_ref, w2_ref, b2_ref,
                                  g_ref, bb_ref, last=False)


def _edge_mlp_last_body(g2_ref, p2_ref, w1_ref, b1_ref, w2_ref, b2_ref,
                        g_ref, bb_ref, upd2_ref, enew_ref):
    e = p2_ref[:, LATENT_C:]
    xin = jnp.concatenate([g2_ref[...], e], axis=-1)
    upd, enew = _edge_core(xin, e, w1_ref, b1_ref, w2_ref, b2_ref,
                           g_ref, bb_ref, last=True)
    upd2_ref[...] = jnp.concatenate([upd, upd], axis=-1)
    enew_ref[...] = enew


def _tc_edge_mlp(kind, g2, e, w1, b1, w2, b2, g, b):
    grid = (N_EDGES_C // _EBLK,)
    wide = pl.BlockSpec((_EBLK, 2 * LATENT_C), lambda i: (i, 0))
    e_spec = pl.BlockSpec((_EBLK, e.shape[1]), lambda i: (i, 0))
    full = lambda a: pl.BlockSpec(a.shape, lambda i: (0,) * a.ndim)
    body = {"first": _edge_mlp_first_body, "mid": _edge_mlp_mid_body,
            "last": _edge_mlp_last_body}[kind]
    wide_out = jax.ShapeDtypeStruct((N_EDGES_C, 2 * LATENT_C), jnp.float32)
    if kind == "last":
        out_specs = [wide, pl.BlockSpec((_EBLK, LATENT_C), lambda i: (i, 0))]
        out_shape = [wide_out,
                     jax.ShapeDtypeStruct((N_EDGES_C, LATENT_C), jnp.float32)]
    else:
        out_specs = wide
        out_shape = wide_out
    return pl.pallas_call(
        body,
        grid=grid,
        in_specs=[wide, e_spec,
                  full(w1), full(b1), full(w2), full(b2), full(g), full(b)],
        out_specs=out_specs,
        out_shape=out_shape,
    )(g2, e, w1, b1, w2, b2, g, b)


_NBLK = 5000


def _node_mlp_body(x_ref, a_ref, w1_ref, b1_ref, w2_ref, b2_ref,
                   g_ref, bb_ref, xnew_ref):
    xin = jnp.concatenate([x_ref[...], a_ref[...]], axis=-1)
    h = jnp.dot(xin, w1_ref[...], preferred_element_type=jnp.float32)
    h = h + b1_ref[...]
    h = h * jax.nn.sigmoid(h)
    o = jnp.dot(h, w2_ref[...], preferred_element_type=jnp.float32)
    o = o + b2_ref[...]
    mu = jnp.mean(o, axis=-1, keepdims=True)
    var = jnp.mean((o - mu) ** 2, axis=-1, keepdims=True)
    on = (o - mu) * lax.rsqrt(var + 1e-5)
    xnew_ref[...] = x_ref[...] + on * g_ref[...] + bb_ref[...]


def _tc_node_mlp(x, agg, w1, b1, w2, b2, g, b):
    grid = (N_NODES_C // _NBLK,)
    row_spec = pl.BlockSpec((_NBLK, LATENT_C), lambda i: (i, 0))
    full = lambda a: pl.BlockSpec(a.shape, lambda i: (0,) * a.ndim)
    return pl.pallas_call(
        _node_mlp_body,
        grid=grid,
        in_specs=[row_spec, row_spec,
                  full(w1), full(b1), full(w2), full(b2), full(g), full(b)],
        out_specs=row_spec,
        out_shape=jax.ShapeDtypeStruct((N_NODES_C, LATENT_C), jnp.float32),
    )(x, agg, w1, b1, w2, b2, g, b)


def kernel(x, edge_index, edge_attr, params):
    src2 = edge_index[0].astype(jnp.int32).reshape(N_EDGES_C // _IDXW, _IDXW)
    dst2 = edge_index[1].astype(jnp.int32).reshape(N_EDGES_C // _IDXW, _IDXW)
    row2 = lambda a: a.reshape(1, -1)
    n_layers = len(params)
    prev2 = None
    e_new = None
    for li, lp in enumerate(params):
        ep, np_ = lp['edge'], lp['node']
        g2 = _sc_gather_kernel()(x, src2, dst2)
        kind = ("first" if li == 0 else
                "last" if li == n_layers - 1 else "mid")
        e_arg = edge_attr if li == 0 else prev2
        res = _tc_edge_mlp(kind, g2, e_arg,
                           ep['W1'], row2(ep['b1']),
                           ep['W2'], row2(ep['b2']),
                           row2(ep['g']), row2(ep['b']))
        if kind == "last":
            upd2, e_new = res
        else:
            upd2 = res
        agg = _sc_scatter_kernel()(upd2, dst2)
        x = _tc_node_mlp(x, agg,
                         np_['W1'], row2(np_['b1']),
                         np_['W2'], row2(np_['b2']),
                         row2(np_['g']), row2(np_['b']))
        prev2 = upd2
    return (x, e_new)
